# baseline (speedup 1.0000x reference)
import jax
import jax.numpy as jnp
from jax import lax
from jax.experimental import pallas as pl
from jax.experimental.pallas import tpu as pltpu
from jax.experimental.pallas import tpu_sc as plsc

N = 10000
D = 128
E = 640000
V = 10000

NC = 2
NS = 16
NW = NC * NS

CH = 256
OPG = 20
NG = 4
EPT = NG * OPG * CH
EPAD = NW * EPT
APAD = 512

XPAD = 10240
XCH = 80
XNCH = XPAD // (NW * XCH)

_mesh = plsc.VectorSubcoreMesh(core_axis_name="c", subcore_axis_name="s")


def _emb_body(emb_hbm, idx_hbm, out_hbm, idx_v, rows_v, sem):
    c = lax.axis_index("c")
    s = lax.axis_index("s")
    wid = s * NC + c
    pltpu.sync_copy(idx_hbm.at[wid], idx_v)

    def step(j, carry):
        pltpu.async_copy(emb_hbm.at[idx_v.at[j]], rows_v, sem).wait()
        pltpu.sync_copy(rows_v, out_hbm.at[pl.ds(wid * (XNCH * XCH) + j * XCH, XCH)])
        return carry

    lax.fori_loop(0, XNCH, step, 0)


def _emb_gather(emb, idx3):
    k = pl.kernel(
        _emb_body,
        out_type=jax.ShapeDtypeStruct((XPAD, D), jnp.float32),
        mesh=_mesh,
        scratch_types=[
            pltpu.VMEM((XNCH, XCH), jnp.int32),
            pltpu.VMEM((XCH, D), jnp.float32),
            pltpu.SemaphoreType.DMA,
        ],
    )
    return k(emb, idx3)


def _gin_body(h_hbm, src_hbm, dst_hbm, out_hbm,
              srcv, dstv, rows0, acc, sem_g):
    c = lax.axis_index("c")
    s = lax.axis_index("s")
    wid = s * NC + c
    rpt = 624
    tail = N - NS * rpt

    pltpu.sync_copy(h_hbm.at[pl.ds(s * rpt, rpt)], acc.at[pl.ds(s * rpt, rpt)])

    @pl.when(s == 0)
    def _():
        pltpu.sync_copy(h_hbm.at[pl.ds(NS * rpt, tail)],
                        acc.at[pl.ds(NS * rpt, tail)])

    plsc.subcore_barrier()

    def group(gi, carry):
        pltpu.sync_copy(src_hbm.at[wid, gi], srcv)
        pltpu.sync_copy(dst_hbm.at[wid, gi], dstv)
        pltpu.async_copy(h_hbm.at[srcv.at[pl.ds(0, CH)]], rows0, sem_g)

        def step(j, carry2):
            pltpu.make_async_copy(
                h_hbm.at[srcv.at[pl.ds(j * CH, CH)]], rows0, sem_g).wait()
            pltpu.sync_copy(rows0, acc.at[dstv.at[pl.ds(j * CH, CH)]], add=True)

            @pl.when(j < OPG - 1)
            def _():
                pltpu.async_copy(
                    h_hbm.at[srcv.at[pl.ds((j + 1) * CH, CH)]], rows0, sem_g)

            return carry2

        lax.fori_loop(0, OPG, step, 0)
        return carry

    lax.fori_loop(0, NG, group, 0)
    plsc.subcore_barrier()
    pltpu.sync_copy(acc.at[pl.ds(s * rpt, rpt)],
                    out_hbm.at[pl.ds(c * N + s * rpt, rpt)])

    @pl.when(s == 0)
    def _():
        pltpu.sync_copy(acc.at[pl.ds(NS * rpt, tail)],
                        out_hbm.at[pl.ds(c * N + NS * rpt, tail)])


def _gin_partials(h, src4, dst4):
    k = pl.kernel(
        _gin_body,
        out_type=jax.ShapeDtypeStruct((NC * N, D), jnp.float32),
        mesh=_mesh,
        scratch_types=[
            pltpu.VMEM((OPG * CH,), jnp.int32),
            pltpu.VMEM((OPG * CH,), jnp.int32),
            pltpu.VMEM((CH, D), jnp.float32),
            pltpu.VMEM_SHARED((N + APAD, D), jnp.float32),
            pltpu.SemaphoreType.DMA,
        ],
    )
    return k(h, src4, dst4)


def _bn_prelu(o, g, be, a):
    m = jnp.mean(o, axis=0, keepdims=True)
    v = jnp.mean((o - m) * (o - m), axis=0, keepdims=True)
    o = (o - m) * lax.rsqrt(v + 1e-5) * g + be
    return jnp.where(o >= 0, o, a * o)


def _mlp_body(p_ref, h_ref, W1_ref, b1_ref, g1_ref, be1_ref, a1_ref,
              W2_ref, b2_ref, g2_ref, be2_ref, a2_ref, out_ref):
    h = h_ref[...]
    g = p_ref[0:N, :] + p_ref[N:2 * N, :] - h
    o = jnp.dot(g, W1_ref[...], preferred_element_type=jnp.float32) + b1_ref[...]
    o = _bn_prelu(o, g1_ref[...], be1_ref[...], a1_ref[...])
    o = jnp.dot(o, W2_ref[...], preferred_element_type=jnp.float32) + b2_ref[...]
    o = _bn_prelu(o, g2_ref[...], be2_ref[...], a2_ref[...])
    out_ref[...] = o + h


def _conv_mlp(parts, h, cp):
    f = pl.pallas_call(
        _mlp_body,
        out_shape=jax.ShapeDtypeStruct((N, D), jnp.float32),
    )
    r = lambda a: a.reshape(1, -1)
    return f(parts, h, cp['W1'], r(cp['b1']), r(cp['g1']), r(cp['be1']),
             cp['a1'].reshape(1, 1), cp['W2'], r(cp['b2']), r(cp['g2']),
             r(cp['be2']), cp['a2'].reshape(1, 1))


def _proj_body(h_ref, Wp_ref, out_ref):
    out_ref[...] = jnp.dot(h_ref[...], Wp_ref[...],
                           preferred_element_type=jnp.float32)


def _proj(h, Wp):
    return pl.pallas_call(
        _proj_body,
        out_shape=jax.ShapeDtypeStruct((N, D), jnp.float32),
    )(h, Wp)


def _lin_body(p_ref, h_ref, Wt_ref, bt_ref, out_ref):
    g = p_ref[0:N, :] + p_ref[N:2 * N, :] - h_ref[...]
    out_ref[...] = jnp.dot(g, Wt_ref[...],
                           preferred_element_type=jnp.float32) + bt_ref[...]


def _lin(parts, h, Wt, bt):
    return pl.pallas_call(
        _lin_body,
        out_shape=jax.ShapeDtypeStruct((N, D), jnp.float32),
    )(parts, h, Wt, bt.reshape(1, -1))


RB = 400


def _big_body(t_ref, W_ref, b_ref, out_ref):
    out_ref[...] = jnp.dot(t_ref[...], W_ref[...],
                           preferred_element_type=jnp.float32) + b_ref[...]


def _big_matmul(t, Wpr, bpr):
    return pl.pallas_call(
        _big_body,
        grid=(N // RB,),
        in_specs=[
            pl.BlockSpec((RB, D), lambda i: (i, 0)),
            pl.BlockSpec((D, V), lambda i: (0, 0)),
            pl.BlockSpec((1, V), lambda i: (0, 0)),
        ],
        out_specs=pl.BlockSpec((RB, V), lambda i: (i, 0)),
        out_shape=jax.ShapeDtypeStruct((N, V), jnp.float32),
    )(t, Wpr, bpr.reshape(1, -1))


def kernel(params, x, edge_index):
    src4 = jnp.concatenate(
        [edge_index[0], jnp.zeros((EPAD - E,), jnp.int32)]
    ).reshape(NW, NG, OPG * CH)
    dst4 = jnp.concatenate(
        [edge_index[1],
         N + jnp.arange(EPAD - E, dtype=jnp.int32) % APAD]
    ).reshape(NW, NG, OPG * CH)
    xpad = jnp.concatenate(
        [x, jnp.zeros((XPAD - N,), jnp.int32)]).reshape(NW, XNCH, XCH)

    h = _emb_gather(params['emb'], xpad)[:N]
    for cp in params['convs']:
        parts = _gin_partials(h, src4, dst4)
        h = _conv_mlp(parts, h, cp)
    h = _proj(h, params['Wp'])
    parts = _gin_partials(h, src4, dst4)
    t = _lin(parts, h, params['Wt'], params['bt'])
    return _big_matmul(t, params['Wpr'], params['bpr'])


# speedup vs baseline: 3.5712x; 3.5712x over previous
import jax
import jax.numpy as jnp
from jax import lax
from jax.experimental import pallas as pl
from jax.experimental.pallas import tpu as pltpu
from jax.experimental.pallas import tpu_sc as plsc

N = 10000
D = 128
E = 640000
V = 10000

NC = 2
NS = 16
NW = NC * NS

CH = 125
G = 40
NG = 4
EPT = NG * G * CH
EPAD = NW * EPT
APAD = 8

XPAD = 10240
XCH = 80
XNCH = XPAD // (NW * XCH)

_mesh = plsc.VectorSubcoreMesh(core_axis_name="c", subcore_axis_name="s")


def _emb_body(emb_hbm, idx_hbm, out_hbm, idx_v, rows_v, sem):
    c = lax.axis_index("c")
    s = lax.axis_index("s")
    wid = s * NC + c
    pltpu.sync_copy(idx_hbm.at[wid], idx_v)

    def step(j, carry):
        pltpu.async_copy(emb_hbm.at[idx_v.at[j]], rows_v, sem).wait()
        pltpu.sync_copy(rows_v, out_hbm.at[pl.ds(wid * (XNCH * XCH) + j * XCH, XCH)])
        return carry

    lax.fori_loop(0, XNCH, step, 0)


def _emb_gather(emb, idx3):
    k = pl.kernel(
        _emb_body,
        out_type=jax.ShapeDtypeStruct((XPAD, D), jnp.float32),
        mesh=_mesh,
        scratch_types=[
            pltpu.VMEM((XNCH, XCH), jnp.int32),
            pltpu.VMEM((XCH, D), jnp.float32),
            pltpu.SemaphoreType.DMA,
        ],
    )
    return k(emb, idx3)


def _gin_body(h_hbm, src_hbm, dst_hbm, out_hbm,
              srcv, dstv, rows0, rows1, acc, sem_g):
    c = lax.axis_index("c")
    s = lax.axis_index("s")
    wid = s * NC + c
    rpt = 624
    tail = N - NS * rpt

    pltpu.sync_copy(h_hbm.at[pl.ds(s * rpt, rpt)], acc.at[pl.ds(s * rpt, rpt)])

    @pl.when(s == 0)
    def _():
        pltpu.sync_copy(h_hbm.at[pl.ds(NS * rpt, tail)],
                        acc.at[pl.ds(NS * rpt, tail)])

    plsc.subcore_barrier()

    def group(gi, carry):
        pltpu.sync_copy(src_hbm.at[wid, gi], srcv)
        pltpu.sync_copy(dst_hbm.at[wid, gi], dstv)
        pltpu.async_copy(h_hbm.at[srcv.at[0]], rows0, sem_g)

        def step(i, carry2):
            j0 = i * 2
            j1 = j0 + 1
            pltpu.make_async_copy(h_hbm.at[srcv.at[j0]], rows0, sem_g).wait()
            pltpu.async_copy(h_hbm.at[srcv.at[j1]], rows1, sem_g)
            pltpu.sync_copy(rows0, acc.at[dstv.at[j0]], add=True)
            pltpu.make_async_copy(h_hbm.at[srcv.at[j1]], rows1, sem_g).wait()

            @pl.when(i < G // 2 - 1)
            def _():
                pltpu.async_copy(h_hbm.at[srcv.at[j0 + 2]], rows0, sem_g)

            pltpu.sync_copy(rows1, acc.at[dstv.at[j1]], add=True)
            return carry2

        lax.fori_loop(0, G // 2, step, 0)
        return carry

    lax.fori_loop(0, NG, group, 0)
    plsc.subcore_barrier()
    pltpu.sync_copy(acc.at[pl.ds(s * rpt, rpt)],
                    out_hbm.at[pl.ds(c * N + s * rpt, rpt)])

    @pl.when(s == 0)
    def _():
        pltpu.sync_copy(acc.at[pl.ds(NS * rpt, tail)],
                        out_hbm.at[pl.ds(c * N + NS * rpt, tail)])


def _gin_partials(h, src4, dst4):
    k = pl.kernel(
        _gin_body,
        out_type=jax.ShapeDtypeStruct((NC * N, D), jnp.float32),
        mesh=_mesh,
        scratch_types=[
            pltpu.VMEM((G, CH), jnp.int32),
            pltpu.VMEM((G, CH), jnp.int32),
            pltpu.VMEM((CH, D), jnp.float32),
            pltpu.VMEM((CH, D), jnp.float32),
            pltpu.VMEM_SHARED((N + APAD, D), jnp.float32),
            pltpu.SemaphoreType.DMA,
        ],
    )
    return k(h, src4, dst4)


def _bn_prelu(o, g, be, a):
    m = jnp.mean(o, axis=0, keepdims=True)
    v = jnp.mean((o - m) * (o - m), axis=0, keepdims=True)
    o = (o - m) * lax.rsqrt(v + 1e-5) * g + be
    return jnp.where(o >= 0, o, a * o)


def _mlp_body(p_ref, h_ref, W1_ref, b1_ref, g1_ref, be1_ref, a1_ref,
              W2_ref, b2_ref, g2_ref, be2_ref, a2_ref, out_ref):
    h = h_ref[...]
    g = p_ref[0:N, :] + p_ref[N:2 * N, :] - h
    o = jnp.dot(g, W1_ref[...], preferred_element_type=jnp.float32) + b1_ref[...]
    o = _bn_prelu(o, g1_ref[...], be1_ref[...], a1_ref[...])
    o = jnp.dot(o, W2_ref[...], preferred_element_type=jnp.float32) + b2_ref[...]
    o = _bn_prelu(o, g2_ref[...], be2_ref[...], a2_ref[...])
    out_ref[...] = o + h


def _conv_mlp(parts, h, cp):
    f = pl.pallas_call(
        _mlp_body,
        out_shape=jax.ShapeDtypeStruct((N, D), jnp.float32),
    )
    r = lambda a: a.reshape(1, -1)
    return f(parts, h, cp['W1'], r(cp['b1']), r(cp['g1']), r(cp['be1']),
             cp['a1'].reshape(1, 1), cp['W2'], r(cp['b2']), r(cp['g2']),
             r(cp['be2']), cp['a2'].reshape(1, 1))


def _proj_body(h_ref, Wp_ref, out_ref):
    out_ref[...] = jnp.dot(h_ref[...], Wp_ref[...],
                           preferred_element_type=jnp.float32)


def _proj(h, Wp):
    return pl.pallas_call(
        _proj_body,
        out_shape=jax.ShapeDtypeStruct((N, D), jnp.float32),
    )(h, Wp)


def _lin_body(p_ref, h_ref, Wt_ref, bt_ref, out_ref):
    g = p_ref[0:N, :] + p_ref[N:2 * N, :] - h_ref[...]
    out_ref[...] = jnp.dot(g, Wt_ref[...],
                           preferred_element_type=jnp.float32) + bt_ref[...]


def _lin(parts, h, Wt, bt):
    return pl.pallas_call(
        _lin_body,
        out_shape=jax.ShapeDtypeStruct((N, D), jnp.float32),
    )(parts, h, Wt, bt.reshape(1, -1))


RB = 400


def _big_body(t_ref, W_ref, b_ref, out_ref):
    out_ref[...] = jnp.dot(t_ref[...], W_ref[...],
                           preferred_element_type=jnp.float32) + b_ref[...]


def _big_matmul(t, Wpr, bpr):
    return pl.pallas_call(
        _big_body,
        grid=(N // RB,),
        in_specs=[
            pl.BlockSpec((RB, D), lambda i: (i, 0)),
            pl.BlockSpec((D, V), lambda i: (0, 0)),
            pl.BlockSpec((1, V), lambda i: (0, 0)),
        ],
        out_specs=pl.BlockSpec((RB, V), lambda i: (i, 0)),
        out_shape=jax.ShapeDtypeStruct((N, V), jnp.float32),
    )(t, Wpr, bpr.reshape(1, -1))


def kernel(params, x, edge_index):
    src4 = edge_index[0].reshape(NW, NG, G, CH)
    dst4 = edge_index[1].reshape(NW, NG, G, CH)
    xpad = jnp.concatenate(
        [x, jnp.zeros((XPAD - N,), jnp.int32)]).reshape(NW, XNCH, XCH)

    h = _emb_gather(params['emb'], xpad)[:N]
    for cp in params['convs']:
        parts = _gin_partials(h, src4, dst4)
        h = _conv_mlp(parts, h, cp)
    h = _proj(h, params['Wp'])
    parts = _gin_partials(h, src4, dst4)
    t = _lin(parts, h, params['Wt'], params['bt'])
    return _big_matmul(t, params['Wpr'], params['bpr'])
